# Initial kernel scaffold; baseline (speedup 1.0000x reference)
import jax
import jax.numpy as jnp
from jax.experimental import pallas as pl


def kernel(features, V, a, W_self, bias, edge_index, edge_type):
    raise NotImplementedError("write your pallas kernel here")


# speedup vs baseline: 43.4678x; 43.4678x over previous
import functools

import jax
import jax.numpy as jnp
from jax import lax
from jax.experimental import pallas as pl
from jax.experimental.pallas import tpu as pltpu
from jax.experimental.pallas import tpu_sc as plsc

N_NODES = 10000
H = 128
OUT = 128
N_RELS = 64
N_BASES = 8
N_EDGES = 320000

NC = 2
NS = 16
NW = NC * NS
EPW = N_EDGES // NW
C = 16
NCH = EPW // C
HCOL = OUT // 2
TILE_SPAN = 640
ZR = 16
RD = 80


def _proj_body(f_ref, v_ref, o_ref):
    o_ref[0] = jnp.dot(f_ref[...], v_ref[0],
                       preferred_element_type=jnp.float32)


def _project(features, V2):
    blk = 400
    grid = (2, N_NODES // blk)
    return pl.pallas_call(
        _proj_body,
        grid=grid,
        in_specs=[
            pl.BlockSpec((blk, H), lambda p, i: (i, 0)),
            pl.BlockSpec((1, H, N_BASES * HCOL), lambda p, i: (p, 0, 0)),
        ],
        out_specs=pl.BlockSpec((1, blk, N_BASES * HCOL), lambda p, i: (p, i, 0)),
        out_shape=jax.ShapeDtypeStruct((2, N_NODES, N_BASES * HCOL),
                                       jnp.float32),
    )(features, V2)


def _combine_body(acc_ref, f_ref, w_ref, b_ref, o_ref):
    s = jnp.dot(f_ref[...], w_ref[...], preferred_element_type=jnp.float32)
    s = s + b_ref[...]
    lo = acc_ref[0, 0] + acc_ref[0, 1]
    hi = acc_ref[1, 0] + acc_ref[1, 1]
    o_ref[:, 0:HCOL] = jnp.maximum(s[:, 0:HCOL] + lo, 0.0)
    o_ref[:, HCOL:OUT] = jnp.maximum(s[:, HCOL:OUT] + hi, 0.0)


def _combine(acc, features, W_self, bias2d):
    blk = 400
    grid = (N_NODES // blk,)
    return pl.pallas_call(
        _combine_body,
        grid=grid,
        in_specs=[
            pl.BlockSpec((2, NC, blk, HCOL), lambda i: (0, 0, i, 0)),
            pl.BlockSpec((blk, H), lambda i: (i, 0)),
            pl.BlockSpec((H, OUT), lambda i: (0, 0)),
            pl.BlockSpec((1, OUT), lambda i: (0, 0)),
        ],
        out_specs=pl.BlockSpec((blk, OUT), lambda i: (i, 0)),
        out_shape=jax.ShapeDtypeStruct((N_NODES, OUT), jnp.float32),
    )(acc, features, W_self, bias2d)


def _sc_edges(hv, a_pad, src3, dst3, et3):
    mesh = plsc.VectorSubcoreMesh(core_axis_name="c", subcore_axis_name="s",
                                  num_cores=NC, num_subcores=NS)

    @functools.partial(
        pl.kernel,
        out_type=jax.ShapeDtypeStruct((2, NC, N_NODES, HCOL), jnp.float32),
        mesh=mesh,
        compiler_params=pltpu.CompilerParams(use_tc_tiling_on_sc=False),
        scratch_types=[
            pltpu.VMEM((NCH, C), jnp.int32),
            pltpu.VMEM((NCH, C), jnp.int32),
            pltpu.VMEM((NCH, C), jnp.int32),
            pltpu.VMEM((C, N_BASES * HCOL), jnp.float32),
            pltpu.VMEM((C, N_BASES * HCOL), jnp.float32),
            pltpu.VMEM((C, HCOL), jnp.float32),
            pltpu.VMEM((C, HCOL), jnp.float32),
            pltpu.VMEM((ZR, HCOL), jnp.float32),
            pltpu.VMEM((N_RELS, 16), jnp.float32),
            pltpu.VMEM_SHARED((N_NODES, HCOL), jnp.float32),
            pltpu.SemaphoreType.DMA,
            pltpu.SemaphoreType.DMA,
        ],
    )
    def body(hv_hbm, a_hbm, src_hbm, dst_hbm, et_hbm, out_hbm,
             src_v, dst_v, et_v, hv0, hv1, msg0, msg1, zbuf, av,
             acc, gs0, gs1):
        cid = lax.axis_index("c")
        sid = lax.axis_index("s")
        wid = sid * NC + cid

        pltpu.sync_copy(src_hbm.at[wid], src_v)
        pltpu.sync_copy(dst_hbm.at[wid], dst_v)
        pltpu.sync_copy(et_hbm.at[wid], et_v)
        pltpu.sync_copy(a_hbm, av)

        hv_slots = (hv0, hv1)
        msg_slots = (msg0, msg1)
        gsems = (gs0, gs1)

        for p in range(2):
            hv_p = hv_hbm.at[p]
            zzero = jnp.zeros((16,), jnp.float32)
            for i in range(ZR):
                for j in range(HCOL // 16):
                    zbuf[i, pl.ds(16 * j, 16)] = zzero
            for t in range(TILE_SPAN // ZR):
                start = sid * TILE_SPAN + t * ZR

                @pl.when(start < N_NODES)
                def _():
                    pltpu.sync_copy(zbuf, acc.at[pl.ds(start, ZR)])
            plsc.subcore_barrier()

            def issue(ci, slot):
                pltpu.async_copy(hv_p.at[src_v.at[ci]], hv_slots[slot],
                                 gsems[slot])

            def wait(ci, slot):
                pltpu.make_async_copy(hv_p.at[src_v.at[ci]], hv_slots[slot],
                                      gsems[slot]).wait()

            def compute(ci, slot):
                hvb = hv_slots[slot]
                msgb = msg_slots[slot]
                etrow = et_v[ci, :]
                for e in range(C):
                    coefv = av[etrow[e], :]
                    for j in range(HCOL // 16):
                        m = coefv[0] * hvb[e, pl.ds(16 * j, 16)]
                        for b in range(1, N_BASES):
                            m = m + coefv[b] * hvb[e, pl.ds(b * HCOL + 16 * j,
                                                            16)]
                        msgb[e, pl.ds(16 * j, 16)] = m

            def scatter(ci, slot):
                pltpu.sync_copy(msg_slots[slot], acc.at[dst_v.at[ci]],
                                add=True)

            issue(0, 0)
            issue(1, 1)

            def step(k, carry):
                c0 = 2 * k
                wait(c0, 0)
                compute(c0, 0)
                scatter(c0, 0)
                issue(c0 + 2, 0)

                c1 = 2 * k + 1
                wait(c1, 1)
                compute(c1, 1)
                scatter(c1, 1)

                @pl.when(k < NCH // 2 - 1)
                def _():
                    issue(c1 + 2, 1)

                return carry

            lax.fori_loop(0, NCH // 2, step, None)
            wait(NCH - 1, 0)
            compute(NCH - 1, 0)
            scatter(NCH - 1, 0)

            plsc.subcore_barrier()
            for t in range(TILE_SPAN // RD):
                start = sid * TILE_SPAN + t * RD

                @pl.when(start < N_NODES)
                def _():
                    pltpu.sync_copy(acc.at[pl.ds(start, RD)],
                                    out_hbm.at[p, cid, pl.ds(start, RD)])
            plsc.subcore_barrier()

    return body(hv, a_pad, src3, dst3, et3)


def kernel(features, V, a, W_self, bias, edge_index, edge_type):
    V2 = (V.transpose(1, 0, 2)
           .reshape(H, N_BASES, 2, HCOL)
           .transpose(2, 0, 1, 3)
           .reshape(2, H, N_BASES * HCOL))
    a_pad = jnp.concatenate(
        [a, jnp.zeros((N_RELS, 16 - N_BASES), jnp.float32)], axis=1)
    src3 = edge_index[0].reshape(NW, NCH, C)
    dst3 = edge_index[1].reshape(NW, NCH, C)
    et3 = edge_type.reshape(NW, NCH, C)

    hv = _project(features, V2)
    acc = _sc_edges(hv, a_pad, src3, dst3, et3)
    return _combine(acc, features, W_self, bias.reshape(1, OUT))


# speedup vs baseline: 45.7366x; 1.0522x over previous
import functools

import jax
import jax.numpy as jnp
from jax import lax
from jax.experimental import pallas as pl
from jax.experimental.pallas import tpu as pltpu
from jax.experimental.pallas import tpu_sc as plsc

N_NODES = 10000
H = 128
OUT = 128
N_RELS = 64
N_BASES = 8
N_EDGES = 320000

NC = 2
NS = 16
NW = NC * NS
EPW = N_EDGES // NW
C = 16
NCH = EPW // C
HCOL = OUT // 2
TILE_SPAN = 640
ZR = 16
RD = 80


def _proj_body(f_ref, v_ref, o_ref):
    o_ref[0] = jnp.dot(f_ref[...], v_ref[0],
                       preferred_element_type=jnp.float32)


def _project(features, V2):
    blk = 400
    grid = (2, N_NODES // blk)
    return pl.pallas_call(
        _proj_body,
        grid=grid,
        in_specs=[
            pl.BlockSpec((blk, H), lambda p, i: (i, 0)),
            pl.BlockSpec((1, H, N_BASES * HCOL), lambda p, i: (p, 0, 0)),
        ],
        out_specs=pl.BlockSpec((1, blk, N_BASES * HCOL), lambda p, i: (p, i, 0)),
        out_shape=jax.ShapeDtypeStruct((2, N_NODES, N_BASES * HCOL),
                                       jnp.float32),
    )(features, V2)


def _combine_body(acc_ref, f_ref, w_ref, b_ref, o_ref):
    s = jnp.dot(f_ref[...], w_ref[...], preferred_element_type=jnp.float32)
    s = s + b_ref[...]
    lo = acc_ref[0, 0] + acc_ref[0, 1]
    hi = acc_ref[1, 0] + acc_ref[1, 1]
    o_ref[:, 0:HCOL] = jnp.maximum(s[:, 0:HCOL] + lo, 0.0)
    o_ref[:, HCOL:OUT] = jnp.maximum(s[:, HCOL:OUT] + hi, 0.0)


def _combine(acc, features, W_self, bias2d):
    blk = 400
    grid = (N_NODES // blk,)
    return pl.pallas_call(
        _combine_body,
        grid=grid,
        in_specs=[
            pl.BlockSpec((2, NC, blk, HCOL), lambda i: (0, 0, i, 0)),
            pl.BlockSpec((blk, H), lambda i: (i, 0)),
            pl.BlockSpec((H, OUT), lambda i: (0, 0)),
            pl.BlockSpec((1, OUT), lambda i: (0, 0)),
        ],
        out_specs=pl.BlockSpec((blk, OUT), lambda i: (i, 0)),
        out_shape=jax.ShapeDtypeStruct((N_NODES, OUT), jnp.float32),
    )(acc, features, W_self, bias2d)


def _sc_edges(hv, a_pad, src3, dst3, et3):
    mesh = plsc.VectorSubcoreMesh(core_axis_name="c", subcore_axis_name="s",
                                  num_cores=NC, num_subcores=NS)

    @functools.partial(
        pl.kernel,
        out_type=jax.ShapeDtypeStruct((2, NC, N_NODES, HCOL), jnp.float32),
        mesh=mesh,
        compiler_params=pltpu.CompilerParams(use_tc_tiling_on_sc=False),
        scratch_types=[
            pltpu.VMEM((NCH, C), jnp.int32),
            pltpu.VMEM((NCH, C), jnp.int32),
            pltpu.VMEM((NCH, C), jnp.int32),
            pltpu.VMEM((C, N_BASES * HCOL), jnp.float32),
            pltpu.VMEM((C, N_BASES * HCOL), jnp.float32),
            pltpu.VMEM((C, HCOL), jnp.float32),
            pltpu.VMEM((C, HCOL), jnp.float32),
            pltpu.VMEM((ZR, HCOL), jnp.float32),
            pltpu.VMEM((N_RELS, 16), jnp.float32),
            pltpu.VMEM_SHARED((N_NODES, HCOL), jnp.float32),
            pltpu.SemaphoreType.DMA,
            pltpu.SemaphoreType.DMA,
            pltpu.SemaphoreType.DMA,
            pltpu.SemaphoreType.DMA,
        ],
    )
    def body(hv_hbm, a_hbm, src_hbm, dst_hbm, et_hbm, out_hbm,
             src_v, dst_v, et_v, hv0, hv1, msg0, msg1, zbuf, av,
             acc, gs0, gs1, ss0, ss1):
        cid = lax.axis_index("c")
        sid = lax.axis_index("s")
        wid = sid * NC + cid

        pltpu.sync_copy(src_hbm.at[wid], src_v)
        pltpu.sync_copy(dst_hbm.at[wid], dst_v)
        pltpu.sync_copy(et_hbm.at[wid], et_v)
        pltpu.sync_copy(a_hbm, av)

        hv_slots = (hv0, hv1)
        msg_slots = (msg0, msg1)
        gsems = (gs0, gs1)
        ssems = (ss0, ss1)

        for p in range(2):
            hv_p = hv_hbm.at[p]
            zzero = jnp.zeros((16,), jnp.float32)
            for i in range(ZR):
                for j in range(HCOL // 16):
                    zbuf[i, pl.ds(16 * j, 16)] = zzero
            for t in range(TILE_SPAN // ZR):
                start = sid * TILE_SPAN + t * ZR

                @pl.when(start < N_NODES)
                def _():
                    pltpu.sync_copy(zbuf, acc.at[pl.ds(start, ZR)])
            plsc.subcore_barrier()

            def issue(ci, slot):
                pltpu.async_copy(hv_p.at[src_v.at[ci]], hv_slots[slot],
                                 gsems[slot])

            def wait(ci, slot):
                pltpu.make_async_copy(hv_p.at[src_v.at[ci]], hv_slots[slot],
                                      gsems[slot]).wait()

            def compute(ci, slot):
                hvb = hv_slots[slot]
                msgb = msg_slots[slot]
                etrow = et_v[ci, :]
                for e in range(C):
                    coefv = av[etrow[e], :]
                    for j in range(HCOL // 16):
                        m = coefv[0] * hvb[e, pl.ds(16 * j, 16)]
                        for b in range(1, N_BASES):
                            m = m + coefv[b] * hvb[e, pl.ds(b * HCOL + 16 * j,
                                                            16)]
                        msgb[e, pl.ds(16 * j, 16)] = m

            def scatter(ci, slot):
                pltpu.async_copy(msg_slots[slot], acc.at[dst_v.at[ci]],
                                 ssems[slot], add=True)

            def scatter_wait(ci, slot):
                pltpu.make_async_copy(msg_slots[slot], acc.at[dst_v.at[ci]],
                                      ssems[slot]).wait()

            issue(0, 0)
            issue(1, 1)

            def step(k, carry):
                c0 = 2 * k
                wait(c0, 0)

                @pl.when(k > 0)
                def _():
                    scatter_wait(c0 - 2, 0)

                compute(c0, 0)
                scatter(c0, 0)
                issue(c0 + 2, 0)

                c1 = 2 * k + 1
                wait(c1, 1)

                @pl.when(k > 0)
                def _():
                    scatter_wait(c1 - 2, 1)

                compute(c1, 1)
                scatter(c1, 1)

                @pl.when(k < NCH // 2 - 1)
                def _():
                    issue(c1 + 2, 1)

                return carry

            lax.fori_loop(0, NCH // 2, step, None)
            wait(NCH - 1, 0)
            scatter_wait(NCH - 3, 0)
            compute(NCH - 1, 0)
            scatter(NCH - 1, 0)
            scatter_wait(NCH - 1, 0)
            scatter_wait(NCH - 2, 1)

            plsc.subcore_barrier()
            for t in range(TILE_SPAN // RD):
                start = sid * TILE_SPAN + t * RD

                @pl.when(start < N_NODES)
                def _():
                    pltpu.sync_copy(acc.at[pl.ds(start, RD)],
                                    out_hbm.at[p, cid, pl.ds(start, RD)])
            plsc.subcore_barrier()

    return body(hv, a_pad, src3, dst3, et3)


def kernel(features, V, a, W_self, bias, edge_index, edge_type):
    V2 = (V.transpose(1, 0, 2)
           .reshape(H, N_BASES, 2, HCOL)
           .transpose(2, 0, 1, 3)
           .reshape(2, H, N_BASES * HCOL))
    a_pad = jnp.concatenate(
        [a, jnp.zeros((N_RELS, 16 - N_BASES), jnp.float32)], axis=1)
    src3 = edge_index[0].reshape(NW, NCH, C)
    dst3 = edge_index[1].reshape(NW, NCH, C)
    et3 = edge_type.reshape(NW, NCH, C)

    hv = _project(features, V2)
    acc = _sc_edges(hv, a_pad, src3, dst3, et3)
    return _combine(acc, features, W_self, bias.reshape(1, OUT))


# speedup vs baseline: 53.7510x; 1.1752x over previous
import functools

import jax
import jax.numpy as jnp
from jax import lax
from jax.experimental import pallas as pl
from jax.experimental.pallas import tpu as pltpu
from jax.experimental.pallas import tpu_sc as plsc

N_NODES = 10000
H = 128
OUT = 128
N_RELS = 64
N_BASES = 8
N_EDGES = 320000

NC = 2
NS = 16
NW = NC * NS
EPW = N_EDGES // NW
C = 16
NCH = EPW // C
HCOL = OUT // 2
TILE_SPAN = 640
ZR = 16
RD = 80


PK = N_BASES * HCOL // 2


def _round_hi16(x):
    xi = jax.lax.bitcast_convert_type(x, jnp.int32)
    return (xi + jnp.int32(0x8000)) & jnp.int32(-65536)


def _proj_body(f_ref, va_ref, vb_ref, o_ref):
    a = jnp.dot(f_ref[...], va_ref[0], preferred_element_type=jnp.float32)
    b = jnp.dot(f_ref[...], vb_ref[0], preferred_element_type=jnp.float32)
    lo = jax.lax.shift_right_logical(_round_hi16(a), 16)
    o_ref[0] = _round_hi16(b) | lo


def _project(features, V2A, V2B):
    blk = 400
    grid = (2, N_NODES // blk)
    return pl.pallas_call(
        _proj_body,
        grid=grid,
        in_specs=[
            pl.BlockSpec((blk, H), lambda p, i: (i, 0)),
            pl.BlockSpec((1, H, PK), lambda p, i: (p, 0, 0)),
            pl.BlockSpec((1, H, PK), lambda p, i: (p, 0, 0)),
        ],
        out_specs=pl.BlockSpec((1, blk, PK), lambda p, i: (p, i, 0)),
        out_shape=jax.ShapeDtypeStruct((2, N_NODES, PK), jnp.int32),
    )(features, V2A, V2B)


def _combine_body(acc_ref, f_ref, w_ref, b_ref, o_ref):
    s = jnp.dot(f_ref[...], w_ref[...], preferred_element_type=jnp.float32)
    s = s + b_ref[...]
    lo = acc_ref[0, 0] + acc_ref[0, 1]
    hi = acc_ref[1, 0] + acc_ref[1, 1]
    o_ref[:, 0:HCOL] = jnp.maximum(s[:, 0:HCOL] + lo, 0.0)
    o_ref[:, HCOL:OUT] = jnp.maximum(s[:, HCOL:OUT] + hi, 0.0)


def _combine(acc, features, W_self, bias2d):
    blk = 400
    grid = (N_NODES // blk,)
    return pl.pallas_call(
        _combine_body,
        grid=grid,
        in_specs=[
            pl.BlockSpec((2, NC, blk, HCOL), lambda i: (0, 0, i, 0)),
            pl.BlockSpec((blk, H), lambda i: (i, 0)),
            pl.BlockSpec((H, OUT), lambda i: (0, 0)),
            pl.BlockSpec((1, OUT), lambda i: (0, 0)),
        ],
        out_specs=pl.BlockSpec((blk, OUT), lambda i: (i, 0)),
        out_shape=jax.ShapeDtypeStruct((N_NODES, OUT), jnp.float32),
    )(acc, features, W_self, bias2d)


def _sc_edges(hv, a_pad, src3, dst3, et3):
    mesh = plsc.VectorSubcoreMesh(core_axis_name="c", subcore_axis_name="s",
                                  num_cores=NC, num_subcores=NS)

    @functools.partial(
        pl.kernel,
        out_type=jax.ShapeDtypeStruct((2, NC, N_NODES, HCOL), jnp.float32),
        mesh=mesh,
        compiler_params=pltpu.CompilerParams(use_tc_tiling_on_sc=False),
        scratch_types=[
            pltpu.VMEM((NCH, C), jnp.int32),
            pltpu.VMEM((NCH, C), jnp.int32),
            pltpu.VMEM((NCH, C), jnp.int32),
            pltpu.VMEM((C, PK), jnp.int32),
            pltpu.VMEM((C, PK), jnp.int32),
            pltpu.VMEM((C, HCOL), jnp.float32),
            pltpu.VMEM((C, HCOL), jnp.float32),
            pltpu.VMEM((ZR, HCOL), jnp.float32),
            pltpu.VMEM((N_RELS, 16), jnp.float32),
            pltpu.VMEM_SHARED((N_NODES, HCOL), jnp.float32),
            pltpu.SemaphoreType.DMA,
            pltpu.SemaphoreType.DMA,
            pltpu.SemaphoreType.DMA,
            pltpu.SemaphoreType.DMA,
        ],
    )
    def body(hv_hbm, a_hbm, src_hbm, dst_hbm, et_hbm, out_hbm,
             src_v, dst_v, et_v, hv0, hv1, msg0, msg1, zbuf, av,
             acc, gs0, gs1, ss0, ss1):
        cid = lax.axis_index("c")
        sid = lax.axis_index("s")
        wid = sid * NC + cid

        pltpu.sync_copy(src_hbm.at[wid], src_v)
        pltpu.sync_copy(dst_hbm.at[wid], dst_v)
        pltpu.sync_copy(et_hbm.at[wid], et_v)
        pltpu.sync_copy(a_hbm, av)

        hv_slots = (hv0, hv1)
        msg_slots = (msg0, msg1)
        gsems = (gs0, gs1)
        ssems = (ss0, ss1)

        for p in range(2):
            hv_p = hv_hbm.at[p]
            zzero = jnp.zeros((16,), jnp.float32)
            for i in range(ZR):
                for j in range(HCOL // 16):
                    zbuf[i, pl.ds(16 * j, 16)] = zzero
            for t in range(TILE_SPAN // ZR):
                start = sid * TILE_SPAN + t * ZR

                @pl.when(start < N_NODES)
                def _():
                    pltpu.sync_copy(zbuf, acc.at[pl.ds(start, ZR)])
            plsc.subcore_barrier()

            def issue(ci, slot):
                pltpu.async_copy(hv_p.at[src_v.at[ci]], hv_slots[slot],
                                 gsems[slot])

            def wait(ci, slot):
                pltpu.make_async_copy(hv_p.at[src_v.at[ci]], hv_slots[slot],
                                      gsems[slot]).wait()

            def compute(ci, slot):
                hvb = hv_slots[slot]
                msgb = msg_slots[slot]
                etrow = et_v[ci, :]
                for e in range(C):
                    coefv = av[etrow[e], :]
                    for g in range(2):
                        mlo = mhi = None
                        for b in range(N_BASES):
                            vi = hvb[e, pl.ds(b * (PK // N_BASES) + 16 * g,
                                              16)]
                            lo = jax.lax.bitcast_convert_type(
                                vi << 16, jnp.float32)
                            hi = jax.lax.bitcast_convert_type(
                                vi & jnp.int32(-65536), jnp.float32)
                            cb = coefv[b]
                            if mlo is None:
                                mlo = cb * lo
                                mhi = cb * hi
                            else:
                                mlo = mlo + cb * lo
                                mhi = mhi + cb * hi
                        msgb[e, pl.ds(16 * g, 16)] = mlo
                        msgb[e, pl.ds(32 + 16 * g, 16)] = mhi

            def scatter(ci, slot):
                pltpu.async_copy(msg_slots[slot], acc.at[dst_v.at[ci]],
                                 ssems[slot], add=True)

            def scatter_wait(ci, slot):
                pltpu.make_async_copy(msg_slots[slot], acc.at[dst_v.at[ci]],
                                      ssems[slot]).wait()

            issue(0, 0)
            issue(1, 1)

            def step(k, carry):
                c0 = 2 * k
                wait(c0, 0)

                @pl.when(k > 0)
                def _():
                    scatter_wait(c0 - 2, 0)

                compute(c0, 0)
                scatter(c0, 0)
                issue(c0 + 2, 0)

                c1 = 2 * k + 1
                wait(c1, 1)

                @pl.when(k > 0)
                def _():
                    scatter_wait(c1 - 2, 1)

                compute(c1, 1)
                scatter(c1, 1)

                @pl.when(k < NCH // 2 - 1)
                def _():
                    issue(c1 + 2, 1)

                return carry

            lax.fori_loop(0, NCH // 2, step, None)
            wait(NCH - 1, 0)
            scatter_wait(NCH - 3, 0)
            compute(NCH - 1, 0)
            scatter(NCH - 1, 0)
            scatter_wait(NCH - 1, 0)
            scatter_wait(NCH - 2, 1)

            plsc.subcore_barrier()
            for t in range(TILE_SPAN // RD):
                start = sid * TILE_SPAN + t * RD

                @pl.when(start < N_NODES)
                def _():
                    pltpu.sync_copy(acc.at[pl.ds(start, RD)],
                                    out_hbm.at[p, cid, pl.ds(start, RD)])
            plsc.subcore_barrier()

    return body(hv, a_pad, src3, dst3, et3)


def kernel(features, V, a, W_self, bias, edge_index, edge_type):
    Vt = (V.transpose(1, 0, 2)
           .reshape(H, N_BASES, 2, 2, 32)
           .transpose(2, 3, 0, 1, 4)
           .reshape(2, 2, H, N_BASES * 32))
    V2A = Vt[:, 0]
    V2B = Vt[:, 1]
    a_pad = jnp.concatenate(
        [a, jnp.zeros((N_RELS, 16 - N_BASES), jnp.float32)], axis=1)
    src3 = edge_index[0].reshape(NW, NCH, C)
    dst3 = edge_index[1].reshape(NW, NCH, C)
    et3 = edge_type.reshape(NW, NCH, C)

    hv = _project(features, V2A, V2B)
    acc = _sc_edges(hv, a_pad, src3, dst3, et3)
    return _combine(acc, features, W_self, bias.reshape(1, OUT))
